# baseline (speedup 1.0000x reference)
import functools

import jax
import jax.numpy as jnp
from jax import lax
from jax.experimental import pallas as pl
from jax.experimental.pallas import tpu as pltpu
from jax.experimental.pallas import tpu_sc as plsc

NC = 2
NS = 16
L = 16
NW = NC * NS
B = 16384
D = 64
BW = B // NW
CHUNK = 128
NCHUNK = BW // CHUNK


def _sc_sims_body(emb_hbm, cidx_hbm, tidx_hbm, out_hbm,
                  cidx_v, tidx_v, crows_v, trows_v, sims_v, sem):
    wid = lax.axis_index("s") * NC + lax.axis_index("c")

    pltpu.sync_copy(cidx_hbm.at[wid], cidx_v)
    pltpu.sync_copy(tidx_hbm.at[wid], tidx_v)

    copies = []
    for j in range(NCHUNK):
        copies.append(pltpu.async_copy(
            emb_hbm.at[cidx_v.at[j]], crows_v.at[pl.ds(j * CHUNK, CHUNK)], sem))
        copies.append(pltpu.async_copy(
            emb_hbm.at[tidx_v.at[j]], trows_v.at[pl.ds(j * CHUNK, CHUNK)], sem))
    for c in copies:
        c.wait()

    iota = lax.iota(jnp.int32, L)

    def group(g, _):
        rows = g * L + iota
        acc = jnp.zeros((L,), jnp.float32)
        for d in range(D):
            col = jnp.full((L,), d, jnp.int32)
            cv = plsc.load_gather(crows_v, [rows, col])
            tv = plsc.load_gather(trows_v, [rows, col])
            acc = acc + cv * tv
        sims_v[pl.ds(g * L, L)] = acc
        return _

    lax.fori_loop(0, BW // L, group, 0)

    pltpu.sync_copy(sims_v, out_hbm.at[wid])


@jax.jit
def _sc_sims(emb_weight, cidx, tidx):
    mesh = plsc.VectorSubcoreMesh(core_axis_name="c", subcore_axis_name="s")
    return pl.kernel(
        _sc_sims_body,
        out_type=jax.ShapeDtypeStruct((NW, BW), jnp.float32),
        mesh=mesh,
        compiler_params=pltpu.CompilerParams(
            use_tc_tiling_on_sc=False, needs_layout_passes=False),
        scratch_types=[
            pltpu.VMEM((NCHUNK, CHUNK), jnp.int32),
            pltpu.VMEM((NCHUNK, CHUNK), jnp.int32),
            pltpu.VMEM((BW, D), jnp.float32),
            pltpu.VMEM((BW, D), jnp.float32),
            pltpu.VMEM((BW,), jnp.float32),
            pltpu.SemaphoreType.DMA,
        ],
    )(emb_weight, cidx, tidx)


def _bce_body(sim_ref, label_ref, out_ref):
    s = sim_ref[...]
    lab = label_ref[...]
    loss = jnp.sum(jnp.maximum(s, 0.0) - s * lab
                   + jnp.log(1.0 + jnp.exp(-jnp.abs(s))))
    out_ref[...] = loss[None, None]


@jax.jit
def _bce(sim2d, label2d):
    out = pl.pallas_call(
        _bce_body,
        out_shape=jax.ShapeDtypeStruct((1, 1), jnp.float32),
    )(sim2d, label2d)
    return out[0, 0]


@jax.jit
def kernel(center_idx, target_idx, label, emb_weight, out_emb_weight):
    del out_emb_weight
    cidx = center_idx.reshape(NW, NCHUNK, CHUNK)
    tidx = target_idx.reshape(NW, NCHUNK, CHUNK)
    sims = _sc_sims(emb_weight, cidx, tidx)
    return _bce(sims.reshape(128, 128), label.reshape(128, 128))
